# baseline (speedup 1.0000x reference)
import functools

import jax
import jax.numpy as jnp
from jax import lax
from jax.experimental import pallas as pl
from jax.experimental.pallas import tpu as pltpu
from jax.experimental.pallas import tpu_sc as plsc

NC = 2
NS = 16
NW = NC * NS
CHUNK = 128
NBUF = 8
PACK = 16384


def _tc_transpose(tableT, *, V, D):
  C = PACK
  nsteps = (V + 2 * C - 1) // (2 * C)

  def body(a_ref, o_ref):
    rr = lax.broadcasted_iota(jnp.int32, (D, D), 0)
    cc = lax.broadcasted_iota(jnp.int32, (D, D), 1)
    ident = (rr == cc).astype(jnp.float32)
    dn = (((0,), (0,)), ((), ()))
    ta = lax.dot_general(a_ref[:, :C], ident, dn,
                         preferred_element_type=jnp.float32)
    tb = lax.dot_general(a_ref[:, C:], ident, dn,
                         preferred_element_type=jnp.float32)
    o_ref[...] = jnp.concatenate([ta, tb], axis=1)

  return pl.pallas_call(
      body,
      grid=(nsteps,),
      in_specs=[pl.BlockSpec((D, 2 * C), lambda j: (0, j))],
      out_specs=pl.BlockSpec((C, 2 * D), lambda j: (j, 0)),
      out_shape=jax.ShapeDtypeStruct((nsteps * C, 2 * D), jnp.float32),
  )(tableT)


def _sc_pool(x_flat, seg, table2d, *, B, H, D):
  bpw = B // NW
  ipw = bpw * H
  nch = ipw // CHUNK

  mesh = plsc.VectorSubcoreMesh(core_axis_name="c", subcore_axis_name="s",
                                num_cores=NC, num_subcores=NS)

  @functools.partial(
      pl.kernel,
      out_type=jax.ShapeDtypeStruct((B, D), jnp.float32),
      mesh=mesh,
      compiler_params=pltpu.CompilerParams(use_tc_tiling_on_sc=False),
      scratch_types=[
          pltpu.VMEM((ipw,), jnp.int32),
          pltpu.VMEM((nch, CHUNK), jnp.int32),
          pltpu.VMEM((NBUF, CHUNK, D), jnp.float32),
          pltpu.VMEM_SHARED((NS * bpw, D), jnp.float32),
          pltpu.SemaphoreType.DMA,
          pltpu.SemaphoreType.DMA,
          pltpu.SemaphoreType.DMA,
          pltpu.SemaphoreType.DMA,
          pltpu.SemaphoreType.DMA,
          pltpu.SemaphoreType.DMA,
          pltpu.SemaphoreType.DMA,
          pltpu.SemaphoreType.DMA,
      ],
  )
  def sc_kernel(x_hbm, seg_hbm, table_hbm, out_hbm,
                idx_v, seg_v, bufs, acc_sh, *gsems):
    c = lax.axis_index("c")
    s = lax.axis_index("s")
    wid = s * NC + c

    pltpu.sync_copy(x_hbm.at[pl.ds(wid * ipw, ipw)], idx_v)
    for k in range(1, NBUF):
      pltpu.async_copy(table_hbm.at[idx_v.at[pl.ds(k * CHUNK, CHUNK)]],
                       bufs.at[k], gsems[k])
    pltpu.sync_copy(seg_hbm.at[s], seg_v)

    zeros = jnp.zeros((16,), jnp.float32)
    def _zero(i, _):
      for q in range(D // 16):
        bufs[0, i, pl.ds(q * 16, 16)] = zeros
      return 0
    lax.fori_loop(0, bpw, _zero, 0)
    pltpu.sync_copy(bufs.at[0, pl.ds(0, bpw), :],
                    acc_sh.at[pl.ds(s * bpw, bpw), :])
    pltpu.async_copy(table_hbm.at[idx_v.at[pl.ds(0, CHUNK)]],
                     bufs.at[0], gsems[0])

    def _step(i, _):
      t0 = i * NBUF
      for k in range(NBUF):
        t = t0 + k
        pltpu.make_async_copy(
            table_hbm.at[idx_v.at[pl.ds(0, CHUNK)]], bufs.at[k],
            gsems[k]).wait()
        pltpu.sync_copy(bufs.at[k], acc_sh.at[seg_v.at[t]], add=True)
        nt = t + NBUF

        @pl.when(nt < nch)
        def _():
          pltpu.async_copy(table_hbm.at[idx_v.at[pl.ds(nt * CHUNK, CHUNK)]],
                           bufs.at[k], gsems[k])
      return 0
    lax.fori_loop(0, nch // NBUF, _step, 0)

    pltpu.sync_copy(acc_sh.at[pl.ds(s * bpw, bpw), :],
                    bufs.at[0, pl.ds(0, bpw), :])
    pltpu.sync_copy(bufs.at[0, pl.ds(0, bpw), :],
                    out_hbm.at[pl.ds(wid * bpw, bpw), :])

  return sc_kernel(x_flat, seg, table2d)


def _tc_dense(pooled_sum, W, b2d, *, B, H, D):
  def body(p_ref, w_ref, b_ref, o_ref):
    acc = jnp.dot(p_ref[...], w_ref[...], preferred_element_type=jnp.float32)
    o_ref[...] = jnp.maximum(acc * (1.0 / H) + b_ref[...], 0.0)

  return pl.pallas_call(
      body,
      out_shape=jax.ShapeDtypeStruct((B, D), jnp.float32),
  )(pooled_sum, W, b2d)


def kernel(x, table, W, b):
  B, H = x.shape
  V, D = table.shape
  bpw = B // NW
  ipw = bpw * H
  nch = ipw // CHUNK

  xi = x.astype(jnp.int32)
  u = xi % (2 * PACK)
  x_flat = ((xi - u) + 2 * (u % PACK) + u // PACK).reshape(-1)

  base = jnp.repeat(jnp.arange(bpw, dtype=jnp.int32), H).reshape(nch, CHUNK)
  seg = base[None] + (jnp.arange(NS, dtype=jnp.int32) * bpw)[:, None, None]

  table_rm = _tc_transpose(table.T, V=V, D=D)
  table2d = table_rm.reshape(table_rm.shape[0] * 2, D)
  pooled_sum = _sc_pool(x_flat, seg, table2d, B=B, H=H, D=D)
  return _tc_dense(pooled_sum, W, b.reshape(1, D), B=B, H=H, D=D)


# speedup vs baseline: 1.0107x; 1.0107x over previous
import functools

import jax
import jax.numpy as jnp
from jax import lax
from jax.experimental import pallas as pl
from jax.experimental.pallas import tpu as pltpu
from jax.experimental.pallas import tpu_sc as plsc

NC = 2
NS = 16
NW = NC * NS
CHUNK = 256
NBUF = 4
PACK = 16384


def _tc_transpose(tableT, *, V, D):
  C = PACK
  nsteps = (V + 2 * C - 1) // (2 * C)

  def body(a_ref, o_ref):
    rr = lax.broadcasted_iota(jnp.int32, (D, D), 0)
    cc = lax.broadcasted_iota(jnp.int32, (D, D), 1)
    ident = (rr == cc).astype(jnp.float32)
    dn = (((0,), (0,)), ((), ()))
    ta = lax.dot_general(a_ref[:, :C], ident, dn,
                         preferred_element_type=jnp.float32)
    tb = lax.dot_general(a_ref[:, C:], ident, dn,
                         preferred_element_type=jnp.float32)
    o_ref[...] = jnp.concatenate([ta, tb], axis=1)

  return pl.pallas_call(
      body,
      grid=(nsteps,),
      in_specs=[pl.BlockSpec((D, 2 * C), lambda j: (0, j))],
      out_specs=pl.BlockSpec((C, 2 * D), lambda j: (j, 0)),
      out_shape=jax.ShapeDtypeStruct((nsteps * C, 2 * D), jnp.float32),
  )(tableT)


def _sc_pool(x_flat, seg, table2d, *, B, H, D):
  bpw = B // NW
  ipw = bpw * H
  nch = ipw // CHUNK

  mesh = plsc.VectorSubcoreMesh(core_axis_name="c", subcore_axis_name="s",
                                num_cores=NC, num_subcores=NS)

  @functools.partial(
      pl.kernel,
      out_type=jax.ShapeDtypeStruct((B, D), jnp.float32),
      mesh=mesh,
      compiler_params=pltpu.CompilerParams(use_tc_tiling_on_sc=False),
      scratch_types=[
          pltpu.VMEM((ipw,), jnp.int32),
          pltpu.VMEM((nch, CHUNK), jnp.int32),
          pltpu.VMEM((NBUF, CHUNK, D), jnp.float32),
          pltpu.VMEM_SHARED((NS * bpw, D), jnp.float32),
          pltpu.SemaphoreType.DMA,
          pltpu.SemaphoreType.DMA,
          pltpu.SemaphoreType.DMA,
          pltpu.SemaphoreType.DMA,
      ],
  )
  def sc_kernel(x_hbm, seg_hbm, table_hbm, out_hbm,
                idx_v, seg_v, bufs, acc_sh, *gsems):
    c = lax.axis_index("c")
    s = lax.axis_index("s")
    wid = s * NC + c

    pltpu.sync_copy(x_hbm.at[pl.ds(wid * ipw, ipw)], idx_v)
    for k in range(1, NBUF):
      pltpu.async_copy(table_hbm.at[idx_v.at[pl.ds(k * CHUNK, CHUNK)]],
                       bufs.at[k], gsems[k])
    pltpu.sync_copy(seg_hbm.at[s], seg_v)

    zeros = jnp.zeros((16,), jnp.float32)
    def _zero(i, _):
      for q in range(D // 16):
        bufs[0, i, pl.ds(q * 16, 16)] = zeros
      return 0
    lax.fori_loop(0, bpw, _zero, 0)
    pltpu.sync_copy(bufs.at[0, pl.ds(0, bpw), :],
                    acc_sh.at[pl.ds(s * bpw, bpw), :])
    pltpu.async_copy(table_hbm.at[idx_v.at[pl.ds(0, CHUNK)]],
                     bufs.at[0], gsems[0])

    def _step(i, _):
      t0 = i * NBUF
      for k in range(NBUF):
        t = t0 + k
        pltpu.make_async_copy(
            table_hbm.at[idx_v.at[pl.ds(0, CHUNK)]], bufs.at[k],
            gsems[k]).wait()
        pltpu.sync_copy(bufs.at[k], acc_sh.at[seg_v.at[t]], add=True)
        nt = t + NBUF

        @pl.when(nt < nch)
        def _():
          pltpu.async_copy(table_hbm.at[idx_v.at[pl.ds(nt * CHUNK, CHUNK)]],
                           bufs.at[k], gsems[k])
      return 0
    lax.fori_loop(0, nch // NBUF, _step, 0)

    pltpu.sync_copy(acc_sh.at[pl.ds(s * bpw, bpw), :],
                    bufs.at[0, pl.ds(0, bpw), :])
    pltpu.sync_copy(bufs.at[0, pl.ds(0, bpw), :],
                    out_hbm.at[pl.ds(wid * bpw, bpw), :])

  return sc_kernel(x_flat, seg, table2d)


def _tc_dense(pooled_sum, W, b2d, *, B, H, D):
  def body(p_ref, w_ref, b_ref, o_ref):
    acc = jnp.dot(p_ref[...], w_ref[...], preferred_element_type=jnp.float32)
    o_ref[...] = jnp.maximum(acc * (1.0 / H) + b_ref[...], 0.0)

  return pl.pallas_call(
      body,
      out_shape=jax.ShapeDtypeStruct((B, D), jnp.float32),
  )(pooled_sum, W, b2d)


def kernel(x, table, W, b):
  B, H = x.shape
  V, D = table.shape
  bpw = B // NW
  ipw = bpw * H
  nch = ipw // CHUNK

  xi = x.astype(jnp.int32)
  u = xi % (2 * PACK)
  x_flat = ((xi - u) + 2 * (u % PACK) + u // PACK).reshape(-1)

  base = jnp.repeat(jnp.arange(bpw, dtype=jnp.int32), H).reshape(nch, CHUNK)
  seg = base[None] + (jnp.arange(NS, dtype=jnp.int32) * bpw)[:, None, None]

  table_rm = _tc_transpose(table.T, V=V, D=D)
  table2d = table_rm.reshape(table_rm.shape[0] * 2, D)
  pooled_sum = _sc_pool(x_flat, seg, table2d, B=B, H=H, D=D)
  return _tc_dense(pooled_sum, W, b.reshape(1, D), B=B, H=H, D=D)
